# baseline (speedup 1.0000x reference)
import functools

import jax
import jax.numpy as jnp
from jax import lax
from jax.experimental import pallas as pl
from jax.experimental.pallas import tpu as pltpu
from jax.experimental.pallas import tpu_sc as plsc

N = 512
C = 256
P = 7
ROWS_PER_ROI = P * P * 2 * 2 * 4
G = N * ROWS_PER_ROI
BINS = N * P * P

_SIDES = (256, 128, 64, 32)
_SCALES = (0.25, 0.125, 0.0625, 0.03125)
_BASES = (0, 2 * 256 * 256, 2 * 256 * 256 + 2 * 128 * 128,
          2 * 256 * 256 + 2 * 128 * 128 + 2 * 64 * 64)
TABLE_ROWS = _BASES[3] + 2 * 32 * 32


def _sel4(lvl, v0, v1, v2, v3, dtype):
    c0 = jnp.asarray(v0, dtype)
    c1 = jnp.asarray(v1, dtype)
    c2 = jnp.asarray(v2, dtype)
    c3 = jnp.asarray(v3, dtype)
    return jnp.where(lvl == 0, c0,
                     jnp.where(lvl == 1, c1, jnp.where(lvl == 2, c2, c3)))


def _prep_body(rois_ref, idx_ref, w_ref):
    rois = rois_ref[...]
    b = rois[:, 0:1].astype(jnp.int32)
    x1 = rois[:, 1:2]
    y1 = rois[:, 2:3]
    x2 = rois[:, 3:4]
    y2 = rois[:, 4:5]

    size = jnp.sqrt((x2 - x1) * (y2 - y1))
    lvlf = jnp.floor(4.0 + jnp.log2(size / 224.0 + 1e-8))
    lvl = jnp.clip(lvlf, 2.0, 5.0).astype(jnp.int32) - 2

    scale = _sel4(lvl, *_SCALES, rois.dtype)
    side_i = _sel4(lvl, *_SIDES, jnp.int32)
    side_f = side_i.astype(rois.dtype)
    rowbase = (_sel4(lvl, *_BASES, jnp.int32)
               + b * _sel4(lvl, *(s * s for s in _SIDES), jnp.int32))

    x1s = x1 * scale - 0.5
    y1s = y1 * scale - 0.5
    bw = (x2 * scale - 0.5 - x1s) / P
    bh = (y2 * scale - 0.5 - y1s) / P

    r = lax.broadcasted_iota(jnp.int32, (1, ROWS_PER_ROI), 1)
    u = r // (P * P)
    py = (r % (P * P)) // P
    px = r % P
    sy = u // 8
    sx = (u // 4) % 2
    tap = u % 4
    fy = py.astype(rois.dtype) + (sy.astype(rois.dtype) + 0.5) * 0.5
    fx = px.astype(rois.dtype) + (sx.astype(rois.dtype) + 0.5) * 0.5

    ys = y1s + fy * bh
    xs = x1s + fx * bw
    valid = ((ys > -1.0) & (ys < side_f) & (xs > -1.0) & (xs < side_f))
    yc = jnp.clip(ys, 0.0, side_f - 1.0)
    xc = jnp.clip(xs, 0.0, side_f - 1.0)
    y0 = yc.astype(jnp.int32)
    x0 = xc.astype(jnp.int32)
    ly = yc - y0.astype(rois.dtype)
    lx = xc - x0.astype(rois.dtype)

    hi_y = tap >= 2
    hi_x = (tap % 2) == 1
    yt = jnp.where(hi_y, jnp.minimum(y0 + 1, side_i - 1), y0)
    xt = jnp.where(hi_x, jnp.minimum(x0 + 1, side_i - 1), x0)
    wy = jnp.where(hi_y, ly, 1.0 - ly)
    wx = jnp.where(hi_x, lx, 1.0 - lx)

    RB = rois.shape[0]
    idx = rowbase + yt * side_i + xt
    w = wy * wx * 0.25 * valid.astype(rois.dtype)
    idx_ref[...] = idx.reshape(RB, 16, P * P).transpose(1, 0, 2)[None]
    w_ref[...] = (w.reshape(RB, 16, P * P).transpose(0, 2, 1)
                  .reshape(RB * P * P, 16))


_PREP_RB = 64
_PREP_CH = 2


def _prep(rois, interpret=False):
    RB = _PREP_RB
    npc = (N // _PREP_CH) // RB
    return pl.pallas_call(
        _prep_body,
        grid=(N // RB,),
        in_specs=[pl.BlockSpec((RB, 5), lambda i: (i, 0))],
        out_specs=[pl.BlockSpec((1, 16, RB, P * P),
                                lambda i: (i // npc, 0, i % npc, 0)),
                   pl.BlockSpec((RB * P * P, 16), lambda i: (i, 0))],
        out_shape=[jax.ShapeDtypeStruct((_PREP_CH, 16, N // _PREP_CH, P * P),
                                        jnp.int32),
                   jax.ShapeDtypeStruct((BINS, 16), jnp.float32)],
        interpret=interpret,
    )(rois)


def _sc_gather(table, idx_flat, win):
    n = idx_flat.shape[0]
    mesh = plsc.VectorSubcoreMesh(core_axis_name="c", subcore_axis_name="s")
    idx2 = idx_flat.reshape(1, n)

    @functools.partial(
        pl.kernel,
        out_type=jax.ShapeDtypeStruct((n, C // 2), table.dtype),
        mesh=mesh,
    )
    def k(table_hbm, idx_hbm, out_hbm):
        def body(i_vmem, o_vmem):
            pltpu.sync_copy(table_hbm.at[i_vmem.at[0]], o_vmem)

        pltpu.emit_pipeline(
            body,
            grid=(n // win,),
            in_specs=[pl.BlockSpec((1, win), index_map=lambda i: (0, i))],
            out_specs=[pl.BlockSpec((win, C // 2), index_map=lambda i: (i, 0))],
            core_axis_name=("c", "s"),
            dimension_semantics=(pltpu.PARALLEL,),
        )(idx_hbm, out_hbm)

    return k(table, idx2)


RN = 8


def _reduce_body(g_ref, w_ref, o_ref):
    g = g_ref[...]
    w = w_ref[...]
    g_lo = lax.bitcast_convert_type(g << 16, jnp.float32)
    g_hi = lax.bitcast_convert_type(g & jnp.int32(-65536), jnp.float32)
    acc_lo = g_lo[0] * w[:, 0:1]
    acc_hi = g_hi[0] * w[:, 0:1]
    for u in range(1, 16):
        wu = w[:, u:u + 1]
        acc_lo = acc_lo + g_lo[u] * wu
        acc_hi = acc_hi + g_hi[u] * wu
    pooled = jnp.concatenate([acc_lo, acc_hi], axis=1)
    pooled = pooled.reshape(RN, P * P, C)
    o_ref[...] = jnp.transpose(pooled, (0, 2, 1))


def _reduce(g3, w2, out, roi0, interpret=False):
    TB = RN * P * P
    nroi = w2.shape[0] // (P * P)
    blk0 = roi0 // RN
    kwargs = {}
    in_specs = [pl.BlockSpec((16, TB, C // 2), lambda i: (0, i, 0)),
                pl.BlockSpec((TB, 16), lambda i: (i, 0))]
    operands = [g3, w2]
    if out is not None:
        in_specs.append(pl.BlockSpec(memory_space=pl.ANY))
        operands.append(out)
        kwargs["input_output_aliases"] = {2: 0}

    def body(*refs):
        _reduce_body(refs[0], refs[1], refs[-1])

    return pl.pallas_call(
        body,
        grid=(nroi // RN,),
        in_specs=in_specs,
        out_specs=pl.BlockSpec((RN, C, P * P), lambda i: (blk0 + i, 0, 0)),
        out_shape=jax.ShapeDtypeStruct((N, C, P * P), jnp.float32),
        interpret=interpret,
        **kwargs,
    )(*operands)


def _pack_body(x_ref, _, o_ref):
    xx = x_ref[0]
    xx = xx.reshape(C, xx.shape[1] * xx.shape[2])
    ai = lax.bitcast_convert_type(
        xx[:C // 2].astype(jnp.bfloat16).astype(jnp.float32), jnp.int32)
    bi = lax.bitcast_convert_type(
        xx[C // 2:].astype(jnp.bfloat16).astype(jnp.float32), jnp.int32)
    packed = lax.shift_right_logical(ai, 16) | bi
    o_ref[...] = jnp.transpose(packed)


def _pack_level(x, table, row0, interpret=False):
    _, _, H, W = x.shape
    RB = min(2048 // W, H)
    PB = RB * W
    grid = (2, H // RB)
    row_blk0 = row0 // PB
    kwargs = {}
    operands = [x]
    if table is None:
        in_specs = [pl.BlockSpec((1, C, RB, W), lambda b, i: (b, 0, i, 0))]
    else:
        in_specs = [pl.BlockSpec((1, C, RB, W), lambda b, i: (b, 0, i, 0)),
                    pl.BlockSpec(memory_space=pl.ANY)]
        operands.append(table)
        kwargs["input_output_aliases"] = {1: 0}
    nyb = H // RB

    def body(*refs):
        if table is None:
            _pack_body(refs[0], None, refs[-1])
        else:
            _pack_body(refs[0], refs[1], refs[-1])

    return pl.pallas_call(
        body,
        grid=grid,
        in_specs=in_specs,
        out_specs=pl.BlockSpec((PB, C // 2),
                               lambda b, i: (row_blk0 + b * nyb + i, 0)),
        out_shape=jax.ShapeDtypeStruct((TABLE_ROWS, C // 2), jnp.int32),
        interpret=interpret,
        **kwargs,
    )(*operands)


def _build_table(x0, x1, x2, x3, interpret=False):
    table = _pack_level(x3, None, _BASES[3], interpret)
    table = _pack_level(x2, table, _BASES[2], interpret)
    table = _pack_level(x1, table, _BASES[1], interpret)
    return _pack_level(x0, table, _BASES[0], interpret)


def kernel(x0, x1, x2, x3, boxes0, boxes1):
    table = _build_table(x0, x1, x2, x3)
    rois = jnp.concatenate([
        jnp.concatenate([jnp.zeros((boxes0.shape[0], 1), boxes0.dtype), boxes0], 1),
        jnp.concatenate([jnp.ones((boxes1.shape[0], 1), boxes1.dtype), boxes1], 1),
    ], 0)
    idx4, w = _prep(rois)
    CH = _PREP_CH
    nroi = N // CH
    nbins = nroi * P * P
    out = None
    for k in range(CH):
        g = _sc_gather(table, idx4[k].reshape(-1), win=128)
        out = _reduce(g.reshape(16, nbins, C // 2),
                      w[k * nbins:(k + 1) * nbins],
                      out, k * nroi)
    return out.reshape(N, C, P, P)


# speedup vs baseline: 1.0295x; 1.0295x over previous
import functools

import jax
import jax.numpy as jnp
from jax import lax
from jax.experimental import pallas as pl
from jax.experimental.pallas import tpu as pltpu
from jax.experimental.pallas import tpu_sc as plsc

N = 512
C = 256
P = 7
ROWS_PER_ROI = P * P * 2 * 2 * 4
G = N * ROWS_PER_ROI
BINS = N * P * P

_SIDES = (256, 128, 64, 32)
_SCALES = (0.25, 0.125, 0.0625, 0.03125)
_BASES = (0, 2 * 256 * 256, 2 * 256 * 256 + 2 * 128 * 128,
          2 * 256 * 256 + 2 * 128 * 128 + 2 * 64 * 64)
TABLE_ROWS = _BASES[3] + 2 * 32 * 32


def _sel4(lvl, v0, v1, v2, v3, dtype):
    c0 = jnp.asarray(v0, dtype)
    c1 = jnp.asarray(v1, dtype)
    c2 = jnp.asarray(v2, dtype)
    c3 = jnp.asarray(v3, dtype)
    return jnp.where(lvl == 0, c0,
                     jnp.where(lvl == 1, c1, jnp.where(lvl == 2, c2, c3)))


def _prep_body(rois_ref, idx_ref, w_ref):
    rois = rois_ref[...]
    b = rois[:, 0:1].astype(jnp.int32)
    x1 = rois[:, 1:2]
    y1 = rois[:, 2:3]
    x2 = rois[:, 3:4]
    y2 = rois[:, 4:5]

    size = jnp.sqrt((x2 - x1) * (y2 - y1))
    lvlf = jnp.floor(4.0 + jnp.log2(size / 224.0 + 1e-8))
    lvl = jnp.clip(lvlf, 2.0, 5.0).astype(jnp.int32) - 2

    scale = _sel4(lvl, *_SCALES, rois.dtype)
    side_i = _sel4(lvl, *_SIDES, jnp.int32)
    side_f = side_i.astype(rois.dtype)
    rowbase = (_sel4(lvl, *_BASES, jnp.int32)
               + b * _sel4(lvl, *(s * s for s in _SIDES), jnp.int32))

    x1s = x1 * scale - 0.5
    y1s = y1 * scale - 0.5
    bw = (x2 * scale - 0.5 - x1s) / P
    bh = (y2 * scale - 0.5 - y1s) / P

    r = lax.broadcasted_iota(jnp.int32, (1, ROWS_PER_ROI), 1)
    u = r // (P * P)
    py = (r % (P * P)) // P
    px = r % P
    sy = u // 8
    sx = (u // 4) % 2
    tap = u % 4
    fy = py.astype(rois.dtype) + (sy.astype(rois.dtype) + 0.5) * 0.5
    fx = px.astype(rois.dtype) + (sx.astype(rois.dtype) + 0.5) * 0.5

    ys = y1s + fy * bh
    xs = x1s + fx * bw
    valid = ((ys > -1.0) & (ys < side_f) & (xs > -1.0) & (xs < side_f))
    yc = jnp.clip(ys, 0.0, side_f - 1.0)
    xc = jnp.clip(xs, 0.0, side_f - 1.0)
    y0 = yc.astype(jnp.int32)
    x0 = xc.astype(jnp.int32)
    ly = yc - y0.astype(rois.dtype)
    lx = xc - x0.astype(rois.dtype)

    hi_y = tap >= 2
    hi_x = (tap % 2) == 1
    yt = jnp.where(hi_y, jnp.minimum(y0 + 1, side_i - 1), y0)
    xt = jnp.where(hi_x, jnp.minimum(x0 + 1, side_i - 1), x0)
    wy = jnp.where(hi_y, ly, 1.0 - ly)
    wx = jnp.where(hi_x, lx, 1.0 - lx)

    RB = rois.shape[0]
    idx = rowbase + yt * side_i + xt
    w = wy * wx * 0.25 * valid.astype(rois.dtype)
    idx_ref[...] = idx.reshape(RB, 16, P * P).transpose(1, 0, 2)[None]
    w_ref[...] = (w.reshape(RB // RN, RN, 16, P * P).transpose(0, 2, 1, 3)
                  .reshape(RB // RN, 16, RN * P * P))


_PREP_RB = 64
_PREP_CH = 2


def _prep(rois, interpret=False):
    RB = _PREP_RB
    npc = (N // _PREP_CH) // RB
    return pl.pallas_call(
        _prep_body,
        grid=(N // RB,),
        in_specs=[pl.BlockSpec((RB, 5), lambda i: (i, 0))],
        out_specs=[pl.BlockSpec((1, 16, RB, P * P),
                                lambda i: (i // npc, 0, i % npc, 0)),
                   pl.BlockSpec((RB // RN, 16, RN * P * P), lambda i: (i, 0, 0))],
        out_shape=[jax.ShapeDtypeStruct((_PREP_CH, 16, N // _PREP_CH, P * P),
                                        jnp.int32),
                   jax.ShapeDtypeStruct((N // RN, 16, RN * P * P), jnp.float32)],
        interpret=interpret,
    )(rois)


def _sc_gather(table, idx_flat, win):
    n = idx_flat.shape[0]
    mesh = plsc.VectorSubcoreMesh(core_axis_name="c", subcore_axis_name="s")
    idx2 = idx_flat.reshape(1, n)

    @functools.partial(
        pl.kernel,
        out_type=jax.ShapeDtypeStruct((n, C // 2), table.dtype),
        mesh=mesh,
    )
    def k(table_hbm, idx_hbm, out_hbm):
        def body(i_vmem, o_vmem):
            pltpu.sync_copy(table_hbm.at[i_vmem.at[0]], o_vmem)

        pltpu.emit_pipeline(
            body,
            grid=(n // win,),
            in_specs=[pl.BlockSpec((1, win), index_map=lambda i: (0, i))],
            out_specs=[pl.BlockSpec((win, C // 2), index_map=lambda i: (i, 0))],
            core_axis_name=("c", "s"),
            dimension_semantics=(pltpu.PARALLEL,),
        )(idx_hbm, out_hbm)

    return k(table, idx2)


RN = 8


def _reduce_body(g_ref, w_ref, o_ref):
    g = g_ref[...]
    w = w_ref[0]
    g_lo = lax.bitcast_convert_type(g << 16, jnp.float32)
    g_hi = lax.bitcast_convert_type(g & jnp.int32(-65536), jnp.float32)
    wj = w[:, :, None]
    pooled = jnp.concatenate(
        [jnp.sum(g_lo * wj, axis=0), jnp.sum(g_hi * wj, axis=0)], axis=1)
    pooled = pooled.reshape(RN, P * P, C)
    o_ref[...] = jnp.transpose(pooled, (0, 2, 1))


def _reduce(g3, w2, out, roi0, interpret=False):
    TB = RN * P * P
    nroi = w2.shape[0] * RN
    blk0 = roi0 // RN
    kwargs = {}
    in_specs = [pl.BlockSpec((16, TB, C // 2), lambda i: (0, i, 0)),
                pl.BlockSpec((1, 16, TB), lambda i: (i, 0, 0))]
    operands = [g3, w2]
    if out is not None:
        in_specs.append(pl.BlockSpec(memory_space=pl.ANY))
        operands.append(out)
        kwargs["input_output_aliases"] = {2: 0}

    def body(*refs):
        _reduce_body(refs[0], refs[1], refs[-1])

    return pl.pallas_call(
        body,
        grid=(nroi // RN,),
        in_specs=in_specs,
        out_specs=pl.BlockSpec((RN, C, P * P), lambda i: (blk0 + i, 0, 0)),
        out_shape=jax.ShapeDtypeStruct((N, C, P * P), jnp.float32),
        interpret=interpret,
        **kwargs,
    )(*operands)


def _pack_body(x_ref, _, o_ref):
    xx = x_ref[0]
    xx = xx.reshape(C, xx.shape[1] * xx.shape[2])
    ai = lax.bitcast_convert_type(
        xx[:C // 2].astype(jnp.bfloat16).astype(jnp.float32), jnp.int32)
    bi = lax.bitcast_convert_type(
        xx[C // 2:].astype(jnp.bfloat16).astype(jnp.float32), jnp.int32)
    packed = lax.shift_right_logical(ai, 16) | bi
    o_ref[...] = jnp.transpose(packed)


def _pack_level(x, table, row0, interpret=False):
    _, _, H, W = x.shape
    RB = min(2048 // W, H)
    PB = RB * W
    grid = (2, H // RB)
    row_blk0 = row0 // PB
    kwargs = {}
    operands = [x]
    if table is None:
        in_specs = [pl.BlockSpec((1, C, RB, W), lambda b, i: (b, 0, i, 0))]
    else:
        in_specs = [pl.BlockSpec((1, C, RB, W), lambda b, i: (b, 0, i, 0)),
                    pl.BlockSpec(memory_space=pl.ANY)]
        operands.append(table)
        kwargs["input_output_aliases"] = {1: 0}
    nyb = H // RB

    def body(*refs):
        if table is None:
            _pack_body(refs[0], None, refs[-1])
        else:
            _pack_body(refs[0], refs[1], refs[-1])

    return pl.pallas_call(
        body,
        grid=grid,
        in_specs=in_specs,
        out_specs=pl.BlockSpec((PB, C // 2),
                               lambda b, i: (row_blk0 + b * nyb + i, 0)),
        out_shape=jax.ShapeDtypeStruct((TABLE_ROWS, C // 2), jnp.int32),
        interpret=interpret,
        **kwargs,
    )(*operands)


def _build_table(x0, x1, x2, x3, interpret=False):
    table = _pack_level(x3, None, _BASES[3], interpret)
    table = _pack_level(x2, table, _BASES[2], interpret)
    table = _pack_level(x1, table, _BASES[1], interpret)
    return _pack_level(x0, table, _BASES[0], interpret)


def kernel(x0, x1, x2, x3, boxes0, boxes1):
    table = _build_table(x0, x1, x2, x3)
    rois = jnp.concatenate([
        jnp.concatenate([jnp.zeros((boxes0.shape[0], 1), boxes0.dtype), boxes0], 1),
        jnp.concatenate([jnp.ones((boxes1.shape[0], 1), boxes1.dtype), boxes1], 1),
    ], 0)
    idx4, w = _prep(rois)
    CH = _PREP_CH
    nroi = N // CH
    nbins = nroi * P * P
    out = None
    for k in range(CH):
        g = _sc_gather(table, idx4[k].reshape(-1), win=128)
        out = _reduce(g.reshape(16, nbins, C // 2),
                      w[k * nroi // RN:(k + 1) * nroi // RN],
                      out, k * nroi)
    return out.reshape(N, C, P, P)


# speedup vs baseline: 1.0531x; 1.0230x over previous
import functools

import jax
import jax.numpy as jnp
from jax import lax
from jax.experimental import pallas as pl
from jax.experimental.pallas import tpu as pltpu
from jax.experimental.pallas import tpu_sc as plsc

N = 512
C = 256
P = 7
ROWS_PER_ROI = P * P * 2 * 2 * 4
G = N * ROWS_PER_ROI
BINS = N * P * P

_SIDES = (256, 128, 64, 32)
_SCALES = (0.25, 0.125, 0.0625, 0.03125)
_BASES = (0, 2 * 256 * 256, 2 * 256 * 256 + 2 * 128 * 128,
          2 * 256 * 256 + 2 * 128 * 128 + 2 * 64 * 64)
TABLE_ROWS = _BASES[3] + 2 * 32 * 32


def _sel4(lvl, v0, v1, v2, v3, dtype):
    c0 = jnp.asarray(v0, dtype)
    c1 = jnp.asarray(v1, dtype)
    c2 = jnp.asarray(v2, dtype)
    c3 = jnp.asarray(v3, dtype)
    return jnp.where(lvl == 0, c0,
                     jnp.where(lvl == 1, c1, jnp.where(lvl == 2, c2, c3)))


def _prep_body(rois_ref, idx_ref, w_ref):
    rois = rois_ref[...]
    b = rois[:, 0:1].astype(jnp.int32)
    x1 = rois[:, 1:2]
    y1 = rois[:, 2:3]
    x2 = rois[:, 3:4]
    y2 = rois[:, 4:5]

    size = jnp.sqrt((x2 - x1) * (y2 - y1))
    lvlf = jnp.floor(4.0 + jnp.log2(size / 224.0 + 1e-8))
    lvl = jnp.clip(lvlf, 2.0, 5.0).astype(jnp.int32) - 2

    scale = _sel4(lvl, *_SCALES, rois.dtype)
    side_i = _sel4(lvl, *_SIDES, jnp.int32)
    side_f = side_i.astype(rois.dtype)
    rowbase = (_sel4(lvl, *_BASES, jnp.int32)
               + b * _sel4(lvl, *(s * s for s in _SIDES), jnp.int32))

    x1s = x1 * scale - 0.5
    y1s = y1 * scale - 0.5
    bw = (x2 * scale - 0.5 - x1s) / P
    bh = (y2 * scale - 0.5 - y1s) / P

    r = lax.broadcasted_iota(jnp.int32, (1, ROWS_PER_ROI), 1)
    u = r // (P * P)
    py = (r % (P * P)) // P
    px = r % P
    sy = u // 8
    sx = (u // 4) % 2
    tap = u % 4
    fy = py.astype(rois.dtype) + (sy.astype(rois.dtype) + 0.5) * 0.5
    fx = px.astype(rois.dtype) + (sx.astype(rois.dtype) + 0.5) * 0.5

    ys = y1s + fy * bh
    xs = x1s + fx * bw
    valid = ((ys > -1.0) & (ys < side_f) & (xs > -1.0) & (xs < side_f))
    yc = jnp.clip(ys, 0.0, side_f - 1.0)
    xc = jnp.clip(xs, 0.0, side_f - 1.0)
    y0 = yc.astype(jnp.int32)
    x0 = xc.astype(jnp.int32)
    ly = yc - y0.astype(rois.dtype)
    lx = xc - x0.astype(rois.dtype)

    hi_y = tap >= 2
    hi_x = (tap % 2) == 1
    yt = jnp.where(hi_y, jnp.minimum(y0 + 1, side_i - 1), y0)
    xt = jnp.where(hi_x, jnp.minimum(x0 + 1, side_i - 1), x0)
    wy = jnp.where(hi_y, ly, 1.0 - ly)
    wx = jnp.where(hi_x, lx, 1.0 - lx)

    RB = rois.shape[0]
    idx = rowbase + yt * side_i + xt
    w = wy * wx * 0.25 * valid.astype(rois.dtype)
    idx_ref[...] = idx.reshape(RB, 16, P * P).transpose(1, 0, 2)[None]
    w_ref[...] = (w.reshape(RB // RN, RN, 16, P * P).transpose(0, 2, 1, 3)
                  .reshape(RB // RN, 16, RN * P * P))


_PREP_RB = 64
_PREP_CH = 2


def _prep(rois, interpret=False):
    RB = _PREP_RB
    npc = (N // _PREP_CH) // RB
    return pl.pallas_call(
        _prep_body,
        grid=(N // RB,),
        in_specs=[pl.BlockSpec((RB, 5), lambda i: (i, 0))],
        out_specs=[pl.BlockSpec((1, 16, RB, P * P),
                                lambda i: (i // npc, 0, i % npc, 0)),
                   pl.BlockSpec((RB // RN, 16, RN * P * P), lambda i: (i, 0, 0))],
        out_shape=[jax.ShapeDtypeStruct((_PREP_CH, 16, N // _PREP_CH, P * P),
                                        jnp.int32),
                   jax.ShapeDtypeStruct((N // RN, 16, RN * P * P), jnp.float32)],
        interpret=interpret,
    )(rois)


def _sc_gather(table, idx_flat, win):
    n = idx_flat.shape[0]
    mesh = plsc.VectorSubcoreMesh(core_axis_name="c", subcore_axis_name="s")
    idx2 = idx_flat.reshape(1, n)

    @functools.partial(
        pl.kernel,
        out_type=jax.ShapeDtypeStruct((n, C // 2), table.dtype),
        mesh=mesh,
    )
    def k(table_hbm, idx_hbm, out_hbm):
        def body(i_vmem, o_vmem):
            pltpu.sync_copy(table_hbm.at[i_vmem.at[0]], o_vmem)

        pltpu.emit_pipeline(
            body,
            grid=(n // win,),
            in_specs=[pl.BlockSpec((1, win), index_map=lambda i: (0, i))],
            out_specs=[pl.BlockSpec((win, C // 2), index_map=lambda i: (i, 0))],
            core_axis_name=("c", "s"),
            dimension_semantics=(pltpu.PARALLEL,),
        )(idx_hbm, out_hbm)

    return k(table, idx2)


RN = 16


def _reduce_body(g_ref, w_ref, o_ref):
    g = g_ref[...]
    w = w_ref[0]
    g_lo = lax.bitcast_convert_type(g << 16, jnp.float32)
    g_hi = lax.bitcast_convert_type(g & jnp.int32(-65536), jnp.float32)
    wj = w[:, :, None]
    pooled = jnp.concatenate(
        [jnp.sum(g_lo * wj, axis=0), jnp.sum(g_hi * wj, axis=0)], axis=1)
    pooled = pooled.reshape(RN, P * P, C)
    o_ref[...] = jnp.transpose(pooled, (0, 2, 1))


def _reduce(g3, w2, out, roi0, interpret=False):
    TB = RN * P * P
    nroi = w2.shape[0] * RN
    blk0 = roi0 // RN
    kwargs = {}
    in_specs = [pl.BlockSpec((16, TB, C // 2), lambda i: (0, i, 0)),
                pl.BlockSpec((1, 16, TB), lambda i: (i, 0, 0))]
    operands = [g3, w2]
    if out is not None:
        in_specs.append(pl.BlockSpec(memory_space=pl.ANY))
        operands.append(out)
        kwargs["input_output_aliases"] = {2: 0}

    def body(*refs):
        _reduce_body(refs[0], refs[1], refs[-1])

    return pl.pallas_call(
        body,
        grid=(nroi // RN,),
        in_specs=in_specs,
        out_specs=pl.BlockSpec((RN, C, P * P), lambda i: (blk0 + i, 0, 0)),
        out_shape=jax.ShapeDtypeStruct((N, C, P * P), jnp.float32),
        interpret=interpret,
        **kwargs,
    )(*operands)


def _pack_body(x_ref, _, o_ref):
    xx = x_ref[0]
    xx = xx.reshape(C, xx.shape[1] * xx.shape[2])
    ai = lax.bitcast_convert_type(
        xx[:C // 2].astype(jnp.bfloat16).astype(jnp.float32), jnp.int32)
    bi = lax.bitcast_convert_type(
        xx[C // 2:].astype(jnp.bfloat16).astype(jnp.float32), jnp.int32)
    packed = lax.shift_right_logical(ai, 16) | bi
    o_ref[...] = jnp.transpose(packed)


def _pack_level(x, table, row0, interpret=False):
    _, _, H, W = x.shape
    RB = min(2048 // W, H)
    PB = RB * W
    grid = (2, H // RB)
    row_blk0 = row0 // PB
    kwargs = {}
    operands = [x]
    if table is None:
        in_specs = [pl.BlockSpec((1, C, RB, W), lambda b, i: (b, 0, i, 0))]
    else:
        in_specs = [pl.BlockSpec((1, C, RB, W), lambda b, i: (b, 0, i, 0)),
                    pl.BlockSpec(memory_space=pl.ANY)]
        operands.append(table)
        kwargs["input_output_aliases"] = {1: 0}
    nyb = H // RB

    def body(*refs):
        if table is None:
            _pack_body(refs[0], None, refs[-1])
        else:
            _pack_body(refs[0], refs[1], refs[-1])

    return pl.pallas_call(
        body,
        grid=grid,
        in_specs=in_specs,
        out_specs=pl.BlockSpec((PB, C // 2),
                               lambda b, i: (row_blk0 + b * nyb + i, 0)),
        out_shape=jax.ShapeDtypeStruct((TABLE_ROWS, C // 2), jnp.int32),
        interpret=interpret,
        **kwargs,
    )(*operands)


def _build_table(x0, x1, x2, x3, interpret=False):
    table = _pack_level(x3, None, _BASES[3], interpret)
    table = _pack_level(x2, table, _BASES[2], interpret)
    table = _pack_level(x1, table, _BASES[1], interpret)
    return _pack_level(x0, table, _BASES[0], interpret)


def kernel(x0, x1, x2, x3, boxes0, boxes1):
    table = _build_table(x0, x1, x2, x3)
    rois = jnp.concatenate([
        jnp.concatenate([jnp.zeros((boxes0.shape[0], 1), boxes0.dtype), boxes0], 1),
        jnp.concatenate([jnp.ones((boxes1.shape[0], 1), boxes1.dtype), boxes1], 1),
    ], 0)
    idx4, w = _prep(rois)
    CH = _PREP_CH
    nroi = N // CH
    nbins = nroi * P * P
    out = None
    for k in range(CH):
        g = _sc_gather(table, idx4[k].reshape(-1), win=128)
        out = _reduce(g.reshape(16, nbins, C // 2),
                      w[k * nroi // RN:(k + 1) * nroi // RN],
                      out, k * nroi)
    return out.reshape(N, C, P, P)


# speedup vs baseline: 1.0607x; 1.0072x over previous
import functools

import jax
import jax.numpy as jnp
from jax import lax
from jax.experimental import pallas as pl
from jax.experimental.pallas import tpu as pltpu
from jax.experimental.pallas import tpu_sc as plsc

N = 512
C = 256
P = 7
ROWS_PER_ROI = P * P * 2 * 2 * 4
G = N * ROWS_PER_ROI
BINS = N * P * P

_SIDES = (256, 128, 64, 32)
_SCALES = (0.25, 0.125, 0.0625, 0.03125)
_BASES = (0, 2 * 256 * 256, 2 * 256 * 256 + 2 * 128 * 128,
          2 * 256 * 256 + 2 * 128 * 128 + 2 * 64 * 64)
TABLE_ROWS = _BASES[3] + 2 * 32 * 32


def _sel4(lvl, v0, v1, v2, v3, dtype):
    c0 = jnp.asarray(v0, dtype)
    c1 = jnp.asarray(v1, dtype)
    c2 = jnp.asarray(v2, dtype)
    c3 = jnp.asarray(v3, dtype)
    return jnp.where(lvl == 0, c0,
                     jnp.where(lvl == 1, c1, jnp.where(lvl == 2, c2, c3)))


def _prep_body(rois_ref, idx_ref, w_ref):
    rois = rois_ref[...]
    b = rois[:, 0:1].astype(jnp.int32)
    x1 = rois[:, 1:2]
    y1 = rois[:, 2:3]
    x2 = rois[:, 3:4]
    y2 = rois[:, 4:5]

    size = jnp.sqrt((x2 - x1) * (y2 - y1))
    lvlf = jnp.floor(4.0 + jnp.log2(size / 224.0 + 1e-8))
    lvl = jnp.clip(lvlf, 2.0, 5.0).astype(jnp.int32) - 2

    scale = _sel4(lvl, *_SCALES, rois.dtype)
    side_i = _sel4(lvl, *_SIDES, jnp.int32)
    side_f = side_i.astype(rois.dtype)
    rowbase = (_sel4(lvl, *_BASES, jnp.int32)
               + b * _sel4(lvl, *(s * s for s in _SIDES), jnp.int32))

    x1s = x1 * scale - 0.5
    y1s = y1 * scale - 0.5
    bw = (x2 * scale - 0.5 - x1s) / P
    bh = (y2 * scale - 0.5 - y1s) / P

    r = lax.broadcasted_iota(jnp.int32, (1, ROWS_PER_ROI), 1)
    u = r // (P * P)
    py = (r % (P * P)) // P
    px = r % P
    sy = u // 8
    sx = (u // 4) % 2
    tap = u % 4
    fy = py.astype(rois.dtype) + (sy.astype(rois.dtype) + 0.5) * 0.5
    fx = px.astype(rois.dtype) + (sx.astype(rois.dtype) + 0.5) * 0.5

    ys = y1s + fy * bh
    xs = x1s + fx * bw
    valid = ((ys > -1.0) & (ys < side_f) & (xs > -1.0) & (xs < side_f))
    yc = jnp.clip(ys, 0.0, side_f - 1.0)
    xc = jnp.clip(xs, 0.0, side_f - 1.0)
    y0 = yc.astype(jnp.int32)
    x0 = xc.astype(jnp.int32)
    ly = yc - y0.astype(rois.dtype)
    lx = xc - x0.astype(rois.dtype)

    hi_y = tap >= 2
    hi_x = (tap % 2) == 1
    yt = jnp.where(hi_y, jnp.minimum(y0 + 1, side_i - 1), y0)
    xt = jnp.where(hi_x, jnp.minimum(x0 + 1, side_i - 1), x0)
    wy = jnp.where(hi_y, ly, 1.0 - ly)
    wx = jnp.where(hi_x, lx, 1.0 - lx)

    RB = rois.shape[0]
    idx = rowbase + yt * side_i + xt
    w = wy * wx * 0.25 * valid.astype(rois.dtype)
    idx_ref[...] = idx.reshape(RB, 16, P * P).transpose(1, 0, 2)[None]
    w_ref[...] = (w.reshape(RB // RN, RN, 16, P * P).transpose(0, 2, 1, 3)
                  .reshape(RB // RN, 16, RN * P * P))


_PREP_RB = 64
_PREP_CH = 2


def _prep(rois, interpret=False):
    RB = _PREP_RB
    npc = (N // _PREP_CH) // RB
    return pl.pallas_call(
        _prep_body,
        grid=(N // RB,),
        in_specs=[pl.BlockSpec((RB, 5), lambda i: (i, 0))],
        out_specs=[pl.BlockSpec((1, 16, RB, P * P),
                                lambda i: (i // npc, 0, i % npc, 0)),
                   pl.BlockSpec((RB // RN, 16, RN * P * P), lambda i: (i, 0, 0))],
        out_shape=[jax.ShapeDtypeStruct((_PREP_CH, 16, N // _PREP_CH, P * P),
                                        jnp.int32),
                   jax.ShapeDtypeStruct((N // RN, 16, RN * P * P), jnp.float32)],
        interpret=interpret,
    )(rois)


def _sc_gather(table, idx_flat, win):
    n = idx_flat.shape[0]
    mesh = plsc.VectorSubcoreMesh(core_axis_name="c", subcore_axis_name="s")
    idx2 = idx_flat.reshape(1, n)

    @functools.partial(
        pl.kernel,
        out_type=jax.ShapeDtypeStruct((n, C // 2), table.dtype),
        mesh=mesh,
    )
    def k(table_hbm, idx_hbm, out_hbm):
        def body(i_vmem, o_vmem):
            pltpu.sync_copy(table_hbm.at[i_vmem.at[0]], o_vmem)

        pltpu.emit_pipeline(
            body,
            grid=(n // win,),
            in_specs=[pl.BlockSpec((1, win), index_map=lambda i: (0, i))],
            out_specs=[pl.BlockSpec((win, C // 2), index_map=lambda i: (i, 0))],
            core_axis_name=("c", "s"),
            dimension_semantics=(pltpu.PARALLEL,),
        )(idx_hbm, out_hbm)

    return k(table, idx2)


RN = 32


def _reduce_body(g_ref, w_ref, o_ref):
    g = g_ref[...]
    w = w_ref[0]
    g_lo = lax.bitcast_convert_type(g << 16, jnp.float32)
    g_hi = lax.bitcast_convert_type(g & jnp.int32(-65536), jnp.float32)
    wj = w[:, :, None]
    pooled = jnp.concatenate(
        [jnp.sum(g_lo * wj, axis=0), jnp.sum(g_hi * wj, axis=0)], axis=1)
    pooled = pooled.reshape(RN, P * P, C)
    o_ref[...] = jnp.transpose(pooled, (0, 2, 1))


def _reduce(g3, w2, out, roi0, interpret=False):
    TB = RN * P * P
    nroi = w2.shape[0] * RN
    blk0 = roi0 // RN
    kwargs = {}
    in_specs = [pl.BlockSpec((16, TB, C // 2), lambda i: (0, i, 0)),
                pl.BlockSpec((1, 16, TB), lambda i: (i, 0, 0))]
    operands = [g3, w2]
    if out is not None:
        in_specs.append(pl.BlockSpec(memory_space=pl.ANY))
        operands.append(out)
        kwargs["input_output_aliases"] = {2: 0}

    def body(*refs):
        _reduce_body(refs[0], refs[1], refs[-1])

    return pl.pallas_call(
        body,
        grid=(nroi // RN,),
        in_specs=in_specs,
        out_specs=pl.BlockSpec((RN, C, P * P), lambda i: (blk0 + i, 0, 0)),
        out_shape=jax.ShapeDtypeStruct((N, C, P * P), jnp.float32),
        interpret=interpret,
        **kwargs,
    )(*operands)


def _pack_body(x_ref, _, o_ref):
    xx = x_ref[0]
    xx = xx.reshape(C, xx.shape[1] * xx.shape[2])
    ai = lax.bitcast_convert_type(
        xx[:C // 2].astype(jnp.bfloat16).astype(jnp.float32), jnp.int32)
    bi = lax.bitcast_convert_type(
        xx[C // 2:].astype(jnp.bfloat16).astype(jnp.float32), jnp.int32)
    packed = lax.shift_right_logical(ai, 16) | bi
    o_ref[...] = jnp.transpose(packed)


def _pack_level(x, table, row0, interpret=False):
    _, _, H, W = x.shape
    RB = min(2048 // W, H)
    PB = RB * W
    grid = (2, H // RB)
    row_blk0 = row0 // PB
    kwargs = {}
    operands = [x]
    if table is None:
        in_specs = [pl.BlockSpec((1, C, RB, W), lambda b, i: (b, 0, i, 0))]
    else:
        in_specs = [pl.BlockSpec((1, C, RB, W), lambda b, i: (b, 0, i, 0)),
                    pl.BlockSpec(memory_space=pl.ANY)]
        operands.append(table)
        kwargs["input_output_aliases"] = {1: 0}
    nyb = H // RB

    def body(*refs):
        if table is None:
            _pack_body(refs[0], None, refs[-1])
        else:
            _pack_body(refs[0], refs[1], refs[-1])

    return pl.pallas_call(
        body,
        grid=grid,
        in_specs=in_specs,
        out_specs=pl.BlockSpec((PB, C // 2),
                               lambda b, i: (row_blk0 + b * nyb + i, 0)),
        out_shape=jax.ShapeDtypeStruct((TABLE_ROWS, C // 2), jnp.int32),
        interpret=interpret,
        **kwargs,
    )(*operands)


def _build_table(x0, x1, x2, x3, interpret=False):
    table = _pack_level(x3, None, _BASES[3], interpret)
    table = _pack_level(x2, table, _BASES[2], interpret)
    table = _pack_level(x1, table, _BASES[1], interpret)
    return _pack_level(x0, table, _BASES[0], interpret)


def kernel(x0, x1, x2, x3, boxes0, boxes1):
    table = _build_table(x0, x1, x2, x3)
    rois = jnp.concatenate([
        jnp.concatenate([jnp.zeros((boxes0.shape[0], 1), boxes0.dtype), boxes0], 1),
        jnp.concatenate([jnp.ones((boxes1.shape[0], 1), boxes1.dtype), boxes1], 1),
    ], 0)
    idx4, w = _prep(rois)
    CH = _PREP_CH
    nroi = N // CH
    nbins = nroi * P * P
    out = None
    for k in range(CH):
        g = _sc_gather(table, idx4[k].reshape(-1), win=128)
        out = _reduce(g.reshape(16, nbins, C // 2),
                      w[k * nroi // RN:(k + 1) * nroi // RN],
                      out, k * nroi)
    return out.reshape(N, C, P, P)
